# baseline (speedup 1.0000x reference)
import functools

import jax
import jax.numpy as jnp
from jax import lax
from jax.experimental import pallas as pl
from jax.experimental.pallas import tpu as pltpu
from jax.experimental.pallas import tpu_sc as plsc

N = 10000
E = 320000
D = 128
DE = 16
H = 10
HP = 16
G = 64

CH = 128
NW = 32
KCW = 80
NCHUNK = KCW * NW
EP = NCHUNK * CH
NP = 10240
RPT = NP // 16

RB = 2048
NB = NP // RB

NBLK = 2000
PL = 512
NPIECE = E // PL
NEB = NCHUNK * CH // (8 * PL)


def _nproj_body(x_ref, w_ref, o_ref):
    o_ref[...] = jnp.dot(x_ref[...], w_ref[...],
                         preferred_element_type=jnp.float32, precision=jax.lax.Precision.HIGHEST)


def _eproj_body(x0, x1, x2, x3, x4, x5, x6, x7, w_ref, b_ref, o_ref):
    dn = (((0,), (0,)), ((), ()))
    parts = [lax.dot_general(x[...], w_ref[...], dn,
                             preferred_element_type=jnp.float32,
                             precision=jax.lax.Precision.HIGHEST)
             for x in (x0, x1, x2, x3, x4, x5, x6, x7)]
    o_ref[...] = jnp.concatenate(parts, axis=1) + b_ref[...]


_sc_mesh = plsc.VectorSubcoreMesh(core_axis_name="c", subcore_axis_name="s")


NBUF = 4
TOUT = KCW // NBUF


@functools.partial(
    pl.kernel,
    mesh=_sc_mesh,
    compiler_params=pltpu.CompilerParams(use_tc_tiling_on_sc=False),
    out_type=jax.ShapeDtypeStruct((2, NP, HP), jnp.float32),
    scratch_types=(
        [pltpu.VMEM((KCW, CH), jnp.int32)] * 2 +
        [pltpu.VMEM((CH // 8, 128), jnp.float32)] * NBUF +
        [pltpu.VMEM((CH, HP), jnp.float32)] * (2 * NBUF) +
        [pltpu.VMEM_SHARED((NP, HP), jnp.float32)] +
        [pltpu.SemaphoreType.DMA] * (3 * NBUF)
    ),
)
def _sc_msg(src_hbm, dst_hbm, eproj_hbm, nproj_hbm, out_hbm, *sc):
    src_v, dst_v = sc[0], sc[1]
    ep = sc[2:2 + NBUF]
    nr = sc[2 + NBUF:2 + 2 * NBUF]
    msg = sc[2 + 2 * NBUF:2 + 3 * NBUF]
    acc_sh = sc[2 + 3 * NBUF]
    esem = sc[3 + 3 * NBUF:3 + 4 * NBUF]
    gsem = sc[3 + 4 * NBUF:3 + 5 * NBUF]
    ssem = sc[3 + 5 * NBUF:3 + 6 * NBUF]

    c = lax.axis_index("c")
    s = lax.axis_index("s")
    wid = s * 2 + c

    def zrow(i, c2):
        msg[0][i, :] = jnp.zeros((HP,), jnp.float32)
        return c2

    lax.fori_loop(0, CH, zrow, 0, unroll=8)

    def zcopy(k, c2):
        pltpu.sync_copy(msg[0], acc_sh.at[pl.ds(s * RPT + k * CH, CH)])
        return c2

    lax.fori_loop(0, RPT // CH, zcopy, 0)
    plsc.subcore_barrier()

    base = wid * KCW
    pltpu.sync_copy(src_hbm.at[pl.ds(base, KCW)], src_v)
    pltpu.sync_copy(dst_hbm.at[pl.ds(base, KCW)], dst_v)

    EPR = CH // 8

    def fetch(q, b):
        pltpu.async_copy(eproj_hbm.at[pl.ds((base + q) * EPR, EPR)],
                         ep[b], esem[b])
        pltpu.async_copy(nproj_hbm.at[src_v.at[q]], nr[b], gsem[b])

    def body(t, b, first, last):
        q = t * NBUF + b
        pltpu.make_async_copy(eproj_hbm.at[pl.ds(0, EPR)], ep[b],
                              esem[b]).wait()
        pltpu.make_async_copy(nproj_hbm.at[src_v.at[0]], nr[b],
                              gsem[b]).wait()
        if not first:
            pltpu.make_async_copy(msg[b], acc_sh.at[dst_v.at[0]],
                                  ssem[b]).wait()

        def row(r, c2):
            for u in range(8):
                i = 8 * r + u
                msg[b][i, :] = jnp.maximum(
                    nr[b][i, :] + ep[b][r, pl.ds(16 * u, 16)], 0.0)
            return c2

        lax.fori_loop(0, CH // 8, row, 0, unroll=2)
        pltpu.async_copy(msg[b], acc_sh.at[dst_v.at[q]], ssem[b], add=True)
        if not last:
            fetch(q + NBUF, b)

    for b in range(NBUF):
        fetch(b, b)
    for b in range(NBUF):
        body(0, b, first=True, last=False)

    def steady(t, carry):
        for b in range(NBUF):
            body(t, b, first=False, last=False)
        return carry

    lax.fori_loop(1, TOUT - 1, steady, 0)
    for b in range(NBUF):
        body(TOUT - 1, b, first=False, last=True)
    for b in range(NBUF):
        pltpu.make_async_copy(msg[b], acc_sh.at[dst_v.at[0]], ssem[b]).wait()

    plsc.subcore_barrier()
    pltpu.sync_copy(acc_sh.at[pl.ds(s * RPT, RPT)],
                    out_hbm.at[c].at[pl.ds(s * RPT, RPT)])


def _post_body(acc_ref, bat_ref, w1_ref, b1_ref, w2_ref, b2_ref,
               w3_ref, b3_ref, w4_ref, b4_ref, o_ref, seg_acc):
    i = pl.program_id(0)
    x = acc_ref[0] + acc_ref[1]
    bid = bat_ref[0, 0, :]
    x = jnp.tanh(jnp.dot(x, w1_ref[...],
                         preferred_element_type=jnp.float32, precision=jax.lax.Precision.HIGHEST) + b1_ref[...])
    x = jnp.tanh(jnp.dot(x, w2_ref[...],
                         preferred_element_type=jnp.float32, precision=jax.lax.Precision.HIGHEST) + b2_ref[...])
    onehot = (bid[None, :] == lax.broadcasted_iota(jnp.int32, (G, RB), 0)
              ).astype(jnp.float32)
    part = jnp.dot(onehot, x, preferred_element_type=jnp.float32, precision=jax.lax.Precision.HIGHEST)

    @pl.when(i == 0)
    def _():
        seg_acc[...] = jnp.zeros_like(seg_acc)

    seg_acc[...] += part

    @pl.when(i == NB - 1)
    def _():
        seg = seg_acc[...]
        y = jnp.tanh(jnp.dot(seg, w3_ref[...],
                             preferred_element_type=jnp.float32, precision=jax.lax.Precision.HIGHEST) + b3_ref[...])
        o_ref[...] = jnp.dot(y, w4_ref[...],
                             preferred_element_type=jnp.float32, precision=jax.lax.Precision.HIGHEST) + b4_ref[...]


def kernel(edge_index, node_attr, edge_attr, batch,
           W_mpl, b_mpl, W1, b1, W2, b2, W3, b3, W4, b4):
    f32 = jnp.float32

    wn = jnp.zeros((D, HP), f32).at[:, :H].set(W_mpl[:D])
    we = jnp.zeros((DE, HP), f32).at[:, :H].set(W_mpl[D:])
    bm = jnp.zeros((1, HP), f32).at[0, :H].set(b_mpl)
    w1p = jnp.zeros((HP, HP), f32).at[:H, :H].set(W1)
    b1p = jnp.zeros((1, HP), f32).at[0, :H].set(b1)
    w2p = jnp.zeros((HP, HP), f32).at[:H, :5].set(W2)
    b2p = jnp.zeros((1, HP), f32).at[0, :5].set(b2)
    w3p = jnp.zeros((HP, HP), f32).at[:5, :5].set(W3)
    b3p = jnp.zeros((1, HP), f32).at[0, :5].set(b3)
    w4p = jnp.zeros((HP, HP), f32).at[:5, :1].set(W4)
    b4p = jnp.zeros((1, HP), f32).at[0, :1].set(b4)

    qa = jnp.arange(NCHUNK, dtype=jnp.int32)[:, None]
    ia = jnp.arange(CH, dtype=jnp.int32)[None, :]
    p = 8 * (qa // 32) + ia % 8
    eidx = PL * jnp.minimum(p, NPIECE - 1) + 16 * (qa % 32) + ia // 8
    src = jnp.take(edge_index[0], eidx)
    dst = jnp.where(p < NPIECE, jnp.take(edge_index[1], eidx), N)

    nproj = pl.pallas_call(
        _nproj_body,
        grid=(N // NBLK,),
        in_specs=[pl.BlockSpec((NBLK, D), lambda i: (i, 0)),
                  pl.BlockSpec((D, HP), lambda i: (0, 0))],
        out_specs=pl.BlockSpec((NBLK, HP), lambda i: (i, 0)),
        out_shape=jax.ShapeDtypeStruct((N, HP), f32),
    )(node_attr, wn)

    bm8 = jnp.tile(bm, (1, 8))
    eat = edge_attr.T
    mkmap = (lambda j: (lambda i: (0, jnp.minimum(8 * i + j, NPIECE - 1))))
    eproj = pl.pallas_call(
        _eproj_body,
        grid=(NEB,),
        in_specs=([pl.BlockSpec((DE, PL), mkmap(j)) for j in range(8)] +
                  [pl.BlockSpec((DE, HP), lambda i: (0, 0)),
                   pl.BlockSpec((1, 128), lambda i: (0, 0))]),
        out_specs=pl.BlockSpec((PL, 128), lambda i: (i, 0)),
        out_shape=jax.ShapeDtypeStruct((NEB * PL, 128), f32),
    )(*([eat] * 8), we, bm8)

    acc = _sc_msg(src, dst, eproj, nproj)

    batp = jnp.concatenate(
        [batch, jnp.full((NP - N,), G, jnp.int32)]).reshape(NB, 1, RB)

    out16 = pl.pallas_call(
        _post_body,
        grid=(NB,),
        in_specs=[pl.BlockSpec((2, RB, HP), lambda i: (0, i, 0)),
                  pl.BlockSpec((1, 1, RB), lambda i: (i, 0, 0)),
                  pl.BlockSpec((HP, HP), lambda i: (0, 0)),
                  pl.BlockSpec((1, HP), lambda i: (0, 0)),
                  pl.BlockSpec((HP, HP), lambda i: (0, 0)),
                  pl.BlockSpec((1, HP), lambda i: (0, 0)),
                  pl.BlockSpec((HP, HP), lambda i: (0, 0)),
                  pl.BlockSpec((1, HP), lambda i: (0, 0)),
                  pl.BlockSpec((HP, HP), lambda i: (0, 0)),
                  pl.BlockSpec((1, HP), lambda i: (0, 0))],
        out_specs=pl.BlockSpec((G, HP), lambda i: (0, 0)),
        out_shape=jax.ShapeDtypeStruct((G, HP), f32),
        scratch_shapes=[pltpu.VMEM((G, HP), f32)],
    )(acc, batp, w1p, b1p, w2p, b2p, w3p, b3p, w4p, b4p)

    return out16[:, :1]


# speedup vs baseline: 1.1078x; 1.1078x over previous
import functools

import jax
import jax.numpy as jnp
from jax import lax
from jax.experimental import pallas as pl
from jax.experimental.pallas import tpu as pltpu
from jax.experimental.pallas import tpu_sc as plsc

N = 10000
E = 320000
D = 128
DE = 16
H = 10
HP = 16
G = 64

CH = 128
NW = 32
KCW = 80
NCHUNK = KCW * NW
EP = NCHUNK * CH
NP = 10240
RPT = NP // 16

RB = 2048
NB = NP // RB

NBLK = 2000
PL = 2560
NPIECE = E // PL
NEB = NCHUNK * CH // (8 * PL)
CPB = PL // 16


def _nproj_body(x_ref, w_ref, o_ref):
    o_ref[...] = jnp.dot(x_ref[...], w_ref[...],
                         preferred_element_type=jnp.float32, precision=jax.lax.Precision.HIGHEST)


def _eproj_body(x0, x1, x2, x3, x4, x5, x6, x7, w_ref, b_ref, o_ref):
    dn = (((0,), (0,)), ((), ()))
    parts = [lax.dot_general(x[...], w_ref[...], dn,
                             preferred_element_type=jnp.float32,
                             precision=jax.lax.Precision.HIGHEST)
             for x in (x0, x1, x2, x3, x4, x5, x6, x7)]
    o_ref[...] = jnp.concatenate(parts, axis=1) + b_ref[...]


_sc_mesh = plsc.VectorSubcoreMesh(core_axis_name="c", subcore_axis_name="s")


NBUF = 4
TOUT = KCW // NBUF


@functools.partial(
    pl.kernel,
    mesh=_sc_mesh,
    compiler_params=pltpu.CompilerParams(use_tc_tiling_on_sc=False),
    out_type=jax.ShapeDtypeStruct((2, NP, HP), jnp.float32),
    scratch_types=(
        [pltpu.VMEM((KCW, CH), jnp.int32)] * 2 +
        [pltpu.VMEM((CH // 8, 128), jnp.float32)] * NBUF +
        [pltpu.VMEM((CH, HP), jnp.float32)] * (2 * NBUF) +
        [pltpu.VMEM_SHARED((NP, HP), jnp.float32)] +
        [pltpu.SemaphoreType.DMA] * (3 * NBUF)
    ),
)
def _sc_msg(src_hbm, dst_hbm, eproj_hbm, nproj_hbm, out_hbm, *sc):
    src_v, dst_v = sc[0], sc[1]
    ep = sc[2:2 + NBUF]
    nr = sc[2 + NBUF:2 + 2 * NBUF]
    msg = sc[2 + 2 * NBUF:2 + 3 * NBUF]
    acc_sh = sc[2 + 3 * NBUF]
    esem = sc[3 + 3 * NBUF:3 + 4 * NBUF]
    gsem = sc[3 + 4 * NBUF:3 + 5 * NBUF]
    ssem = sc[3 + 5 * NBUF:3 + 6 * NBUF]

    c = lax.axis_index("c")
    s = lax.axis_index("s")
    wid = s * 2 + c

    def zrow(i, c2):
        msg[0][i, :] = jnp.zeros((HP,), jnp.float32)
        return c2

    lax.fori_loop(0, CH, zrow, 0, unroll=8)

    def zcopy(k, c2):
        pltpu.sync_copy(msg[0], acc_sh.at[pl.ds(s * RPT + k * CH, CH)])
        return c2

    lax.fori_loop(0, RPT // CH, zcopy, 0)
    plsc.subcore_barrier()

    base = wid * KCW
    pltpu.sync_copy(src_hbm.at[pl.ds(base, KCW)], src_v)
    pltpu.sync_copy(dst_hbm.at[pl.ds(base, KCW)], dst_v)

    EPR = CH // 8

    def fetch(q, b):
        pltpu.async_copy(eproj_hbm.at[pl.ds((base + q) * EPR, EPR)],
                         ep[b], esem[b])
        pltpu.async_copy(nproj_hbm.at[src_v.at[q]], nr[b], gsem[b])

    def body(t, b, first, last):
        q = t * NBUF + b
        pltpu.make_async_copy(eproj_hbm.at[pl.ds(0, EPR)], ep[b],
                              esem[b]).wait()
        pltpu.make_async_copy(nproj_hbm.at[src_v.at[0]], nr[b],
                              gsem[b]).wait()
        if not first:
            pltpu.make_async_copy(msg[b], acc_sh.at[dst_v.at[0]],
                                  ssem[b]).wait()

        def row(r, c2):
            for u in range(8):
                i = 8 * r + u
                msg[b][i, :] = jnp.maximum(
                    nr[b][i, :] + ep[b][r, pl.ds(16 * u, 16)], 0.0)
            return c2

        lax.fori_loop(0, CH // 8, row, 0, unroll=2)
        pltpu.async_copy(msg[b], acc_sh.at[dst_v.at[q]], ssem[b], add=True)
        if not last:
            fetch(q + NBUF, b)

    for b in range(NBUF):
        fetch(b, b)
    for b in range(NBUF):
        body(0, b, first=True, last=False)

    def steady(t, carry):
        for b in range(NBUF):
            body(t, b, first=False, last=False)
        return carry

    lax.fori_loop(1, TOUT - 1, steady, 0)
    for b in range(NBUF):
        body(TOUT - 1, b, first=False, last=True)
    for b in range(NBUF):
        pltpu.make_async_copy(msg[b], acc_sh.at[dst_v.at[0]], ssem[b]).wait()

    plsc.subcore_barrier()
    pltpu.sync_copy(acc_sh.at[pl.ds(s * RPT, RPT)],
                    out_hbm.at[c].at[pl.ds(s * RPT, RPT)])


def _post_body(acc_ref, bat_ref, w1_ref, b1_ref, w2_ref, b2_ref,
               w3_ref, b3_ref, w4_ref, b4_ref, o_ref, seg_acc):
    i = pl.program_id(0)
    x = acc_ref[0] + acc_ref[1]
    bid = bat_ref[0, 0, :]
    x = jnp.tanh(jnp.dot(x, w1_ref[...],
                         preferred_element_type=jnp.float32, precision=jax.lax.Precision.HIGHEST) + b1_ref[...])
    x = jnp.tanh(jnp.dot(x, w2_ref[...],
                         preferred_element_type=jnp.float32, precision=jax.lax.Precision.HIGHEST) + b2_ref[...])
    onehot = (bid[None, :] == lax.broadcasted_iota(jnp.int32, (G, RB), 0)
              ).astype(jnp.float32)
    part = jnp.dot(onehot, x, preferred_element_type=jnp.float32, precision=jax.lax.Precision.HIGHEST)

    @pl.when(i == 0)
    def _():
        seg_acc[...] = jnp.zeros_like(seg_acc)

    seg_acc[...] += part

    @pl.when(i == NB - 1)
    def _():
        seg = seg_acc[...]
        y = jnp.tanh(jnp.dot(seg, w3_ref[...],
                             preferred_element_type=jnp.float32, precision=jax.lax.Precision.HIGHEST) + b3_ref[...])
        o_ref[...] = jnp.dot(y, w4_ref[...],
                             preferred_element_type=jnp.float32, precision=jax.lax.Precision.HIGHEST) + b4_ref[...]


def kernel(edge_index, node_attr, edge_attr, batch,
           W_mpl, b_mpl, W1, b1, W2, b2, W3, b3, W4, b4):
    f32 = jnp.float32

    wn = jnp.zeros((D, HP), f32).at[:, :H].set(W_mpl[:D])
    we = jnp.zeros((DE, HP), f32).at[:, :H].set(W_mpl[D:])
    bm = jnp.zeros((1, HP), f32).at[0, :H].set(b_mpl)
    w1p = jnp.zeros((HP, HP), f32).at[:H, :H].set(W1)
    b1p = jnp.zeros((1, HP), f32).at[0, :H].set(b1)
    w2p = jnp.zeros((HP, HP), f32).at[:H, :5].set(W2)
    b2p = jnp.zeros((1, HP), f32).at[0, :5].set(b2)
    w3p = jnp.zeros((HP, HP), f32).at[:5, :5].set(W3)
    b3p = jnp.zeros((1, HP), f32).at[0, :5].set(b3)
    w4p = jnp.zeros((HP, HP), f32).at[:5, :1].set(W4)
    b4p = jnp.zeros((1, HP), f32).at[0, :1].set(b4)

    qa = jnp.arange(NCHUNK, dtype=jnp.int32)[:, None]
    ia = jnp.arange(CH, dtype=jnp.int32)[None, :]
    p = 8 * (qa // CPB) + ia % 8
    eidx = PL * jnp.minimum(p, NPIECE - 1) + 16 * (qa % CPB) + ia // 8
    src = jnp.take(edge_index[0], eidx)
    dst = jnp.where(p < NPIECE, jnp.take(edge_index[1], eidx), N)

    nproj = pl.pallas_call(
        _nproj_body,
        grid=(N // NBLK,),
        in_specs=[pl.BlockSpec((NBLK, D), lambda i: (i, 0)),
                  pl.BlockSpec((D, HP), lambda i: (0, 0))],
        out_specs=pl.BlockSpec((NBLK, HP), lambda i: (i, 0)),
        out_shape=jax.ShapeDtypeStruct((N, HP), f32),
    )(node_attr, wn)

    bm8 = jnp.tile(bm, (1, 8))
    eat = edge_attr.T
    mkmap = (lambda j: (lambda i: (0, jnp.minimum(8 * i + j, NPIECE - 1))))
    eproj = pl.pallas_call(
        _eproj_body,
        grid=(NEB,),
        in_specs=([pl.BlockSpec((DE, PL), mkmap(j)) for j in range(8)] +
                  [pl.BlockSpec((DE, HP), lambda i: (0, 0)),
                   pl.BlockSpec((1, 128), lambda i: (0, 0))]),
        out_specs=pl.BlockSpec((PL, 128), lambda i: (i, 0)),
        out_shape=jax.ShapeDtypeStruct((NEB * PL, 128), f32),
    )(*([eat] * 8), we, bm8)

    acc = _sc_msg(src, dst, eproj, nproj)

    batp = jnp.concatenate(
        [batch, jnp.full((NP - N,), G, jnp.int32)]).reshape(NB, 1, RB)

    out16 = pl.pallas_call(
        _post_body,
        grid=(NB,),
        in_specs=[pl.BlockSpec((2, RB, HP), lambda i: (0, i, 0)),
                  pl.BlockSpec((1, 1, RB), lambda i: (i, 0, 0)),
                  pl.BlockSpec((HP, HP), lambda i: (0, 0)),
                  pl.BlockSpec((1, HP), lambda i: (0, 0)),
                  pl.BlockSpec((HP, HP), lambda i: (0, 0)),
                  pl.BlockSpec((1, HP), lambda i: (0, 0)),
                  pl.BlockSpec((HP, HP), lambda i: (0, 0)),
                  pl.BlockSpec((1, HP), lambda i: (0, 0)),
                  pl.BlockSpec((HP, HP), lambda i: (0, 0)),
                  pl.BlockSpec((1, HP), lambda i: (0, 0))],
        out_specs=pl.BlockSpec((G, HP), lambda i: (0, 0)),
        out_shape=jax.ShapeDtypeStruct((G, HP), f32),
        scratch_shapes=[pltpu.VMEM((G, HP), f32)],
    )(acc, batp, w1p, b1p, w2p, b2p, w3p, b3p, w4p, b4p)

    return out16[:, :1]


# speedup vs baseline: 1.2242x; 1.1051x over previous
import functools

import jax
import jax.numpy as jnp
from jax import lax
from jax.experimental import pallas as pl
from jax.experimental.pallas import tpu as pltpu
from jax.experimental.pallas import tpu_sc as plsc

N = 10000
E = 320000
D = 128
DE = 16
H = 10
HP = 16
G = 64

CH = 128
NW = 32
KCW = 80
NCHUNK = KCW * NW
EP = NCHUNK * CH
NP = 10240
RPT = NP // 16

RB = 2048
NB = NP // RB

NBLK = 2000
EBLK = 16000
DR = EBLK // 8
G_ROWS = 400
QPB = DR // 16
NCR = E // CH


def _nproj_body(x_ref, w_ref, o_ref):
    o_ref[...] = jnp.dot(x_ref[...], w_ref[...],
                         preferred_element_type=jnp.float32, precision=jax.lax.Precision.HIGHEST)


def _eproj_body(x_ref, w_ref, b_ref, o_ref):
    for g in range(DR // G_ROWS):
        parts = [x_ref[pl.ds(DR * j + G_ROWS * g, G_ROWS), :]
                 for j in range(8)]
        x128 = jnp.concatenate(parts, axis=1)
        y = jnp.dot(x128, w_ref[...], preferred_element_type=jnp.float32,
                    precision=jax.lax.Precision.HIGHEST) + b_ref[...]
        o_ref[pl.ds(G_ROWS * g, G_ROWS), :] = y


_sc_mesh = plsc.VectorSubcoreMesh(core_axis_name="c", subcore_axis_name="s")


NBUF = 4
TOUT = KCW // NBUF


@functools.partial(
    pl.kernel,
    mesh=_sc_mesh,
    compiler_params=pltpu.CompilerParams(use_tc_tiling_on_sc=False),
    out_type=jax.ShapeDtypeStruct((2, NP, HP), jnp.float32),
    scratch_types=(
        [pltpu.VMEM((KCW, CH), jnp.int32)] * 2 +
        [pltpu.VMEM((CH // 8, 128), jnp.float32)] * NBUF +
        [pltpu.VMEM((CH, HP), jnp.float32)] * (2 * NBUF) +
        [pltpu.VMEM_SHARED((NP, HP), jnp.float32)] +
        [pltpu.SemaphoreType.DMA] * (3 * NBUF)
    ),
)
def _sc_msg(src_hbm, dst_hbm, eproj_hbm, nproj_hbm, out_hbm, *sc):
    src_v, dst_v = sc[0], sc[1]
    ep = sc[2:2 + NBUF]
    nr = sc[2 + NBUF:2 + 2 * NBUF]
    msg = sc[2 + 2 * NBUF:2 + 3 * NBUF]
    acc_sh = sc[2 + 3 * NBUF]
    esem = sc[3 + 3 * NBUF:3 + 4 * NBUF]
    gsem = sc[3 + 4 * NBUF:3 + 5 * NBUF]
    ssem = sc[3 + 5 * NBUF:3 + 6 * NBUF]

    c = lax.axis_index("c")
    s = lax.axis_index("s")
    wid = s * 2 + c

    def zrow(i, c2):
        msg[0][i, :] = jnp.zeros((HP,), jnp.float32)
        return c2

    lax.fori_loop(0, CH, zrow, 0, unroll=8)

    def zcopy(k, c2):
        pltpu.sync_copy(msg[0], acc_sh.at[pl.ds(s * RPT + k * CH, CH)])
        return c2

    lax.fori_loop(0, RPT // CH, zcopy, 0)
    plsc.subcore_barrier()

    base = wid * KCW
    pltpu.sync_copy(src_hbm.at[pl.ds(base, KCW)], src_v)
    pltpu.sync_copy(dst_hbm.at[pl.ds(base, KCW)], dst_v)

    EPR = CH // 8

    def fetch(q, b):
        off = jnp.minimum((base + q) * EPR, (NCR - 1) * EPR)
        pltpu.async_copy(eproj_hbm.at[pl.ds(off, EPR)], ep[b], esem[b])
        pltpu.async_copy(nproj_hbm.at[src_v.at[q]], nr[b], gsem[b])

    def body(t, b, first, last):
        q = t * NBUF + b
        pltpu.make_async_copy(eproj_hbm.at[pl.ds(0, EPR)], ep[b],
                              esem[b]).wait()
        pltpu.make_async_copy(nproj_hbm.at[src_v.at[0]], nr[b],
                              gsem[b]).wait()
        if not first:
            pltpu.make_async_copy(msg[b], acc_sh.at[dst_v.at[0]],
                                  ssem[b]).wait()

        def row(r, c2):
            for u in range(8):
                i = 8 * r + u
                msg[b][i, :] = jnp.maximum(
                    nr[b][i, :] + ep[b][r, pl.ds(16 * u, 16)], 0.0)
            return c2

        lax.fori_loop(0, CH // 8, row, 0, unroll=2)
        pltpu.async_copy(msg[b], acc_sh.at[dst_v.at[q]], ssem[b], add=True)
        if not last:
            fetch(q + NBUF, b)

    for b in range(NBUF):
        fetch(b, b)
    for b in range(NBUF):
        body(0, b, first=True, last=False)

    def steady(t, carry):
        for b in range(NBUF):
            body(t, b, first=False, last=False)
        return carry

    lax.fori_loop(1, TOUT - 1, steady, 0)
    for b in range(NBUF):
        body(TOUT - 1, b, first=False, last=True)
    for b in range(NBUF):
        pltpu.make_async_copy(msg[b], acc_sh.at[dst_v.at[0]], ssem[b]).wait()

    plsc.subcore_barrier()
    pltpu.sync_copy(acc_sh.at[pl.ds(s * RPT, RPT)],
                    out_hbm.at[c].at[pl.ds(s * RPT, RPT)])


def _post_body(acc_ref, bat_ref, w1_ref, b1_ref, w2_ref, b2_ref,
               w3_ref, b3_ref, w4_ref, b4_ref, o_ref):
    hp = jnp.float32
    x = acc_ref[0] + acc_ref[1]
    x = jnp.tanh(jnp.dot(x, w1_ref[...],
                         preferred_element_type=hp,
                         precision=jax.lax.Precision.HIGHEST) + b1_ref[...])
    x = jnp.tanh(jnp.dot(x, w2_ref[...],
                         preferred_element_type=hp,
                         precision=jax.lax.Precision.HIGHEST) + b2_ref[...])
    giota = lax.broadcasted_iota(jnp.int32, (G, NP // 8), 0)
    selr = lax.broadcasted_iota(jnp.int32, (128, HP), 0)
    selh = lax.broadcasted_iota(jnp.int32, (128, HP), 1)
    seg = jnp.zeros((G, HP), hp)
    for j in range(8):
        ohj = (bat_ref[j, :][None, :] == giota).astype(hp)
        mj = jnp.dot(ohj, x, preferred_element_type=hp,
                     precision=jax.lax.Precision.HIGHEST)
        selj = (selr == selh + 16 * j).astype(hp)
        seg = seg + jnp.dot(mj, selj, preferred_element_type=hp,
                            precision=jax.lax.Precision.HIGHEST)
    y = jnp.tanh(jnp.dot(seg, w3_ref[...], preferred_element_type=hp,
                         precision=jax.lax.Precision.HIGHEST) + b3_ref[...])
    o_ref[...] = jnp.dot(y, w4_ref[...], preferred_element_type=hp,
                         precision=jax.lax.Precision.HIGHEST) + b4_ref[...]


def kernel(edge_index, node_attr, edge_attr, batch,
           W_mpl, b_mpl, W1, b1, W2, b2, W3, b3, W4, b4):
    f32 = jnp.float32

    wn = jnp.zeros((D, HP), f32).at[:, :H].set(W_mpl[:D])
    we = jnp.zeros((DE, HP), f32).at[:, :H].set(W_mpl[D:])
    bm = jnp.zeros((1, HP), f32).at[0, :H].set(b_mpl)
    w1p = jnp.zeros((HP, HP), f32).at[:H, :H].set(W1)
    b1p = jnp.zeros((1, HP), f32).at[0, :H].set(b1)
    w2p = jnp.zeros((HP, HP), f32).at[:H, :5].set(W2)
    b2p = jnp.zeros((1, HP), f32).at[0, :5].set(b2)
    w3p = jnp.zeros((HP, HP), f32).at[:5, :5].set(W3)
    b3p = jnp.zeros((1, HP), f32).at[0, :5].set(b3)
    w4p = jnp.zeros((HP, HP), f32).at[:5, :1].set(W4)
    b4p = jnp.zeros((1, HP), f32).at[0, :1].set(b4)

    qa = jnp.arange(NCHUNK, dtype=jnp.int32)[:, None]
    ia = jnp.arange(CH, dtype=jnp.int32)[None, :]
    qc = jnp.minimum(qa, NCR - 1)
    eidx = (EBLK * (qc // QPB) + DR * (ia % 8)
            + 16 * (qc % QPB) + ia // 8)
    src = jnp.take(edge_index[0], eidx)
    dst = jnp.where(qa < NCR, jnp.take(edge_index[1], eidx), N)

    nproj = pl.pallas_call(
        _nproj_body,
        grid=(N // NBLK,),
        in_specs=[pl.BlockSpec((NBLK, D), lambda i: (i, 0)),
                  pl.BlockSpec((D, HP), lambda i: (0, 0))],
        out_specs=pl.BlockSpec((NBLK, HP), lambda i: (i, 0)),
        out_shape=jax.ShapeDtypeStruct((N, HP), f32),
    )(node_attr, wn)

    we8 = jnp.kron(jnp.eye(8, dtype=f32), we)
    bm8 = jnp.tile(bm, (1, 8))
    eproj = pl.pallas_call(
        _eproj_body,
        grid=(E // EBLK,),
        in_specs=[pl.BlockSpec((EBLK, DE), lambda i: (i, 0)),
                  pl.BlockSpec((128, 128), lambda i: (0, 0)),
                  pl.BlockSpec((1, 128), lambda i: (0, 0))],
        out_specs=pl.BlockSpec((DR, 128), lambda i: (i, 0)),
        out_shape=jax.ShapeDtypeStruct((E // 8, 128), f32),
    )(edge_attr, we8, bm8)

    acc = _sc_msg(src, dst, eproj, nproj)
    acc2 = acc.reshape(2, NP // 8, 128)

    bat2 = jnp.concatenate(
        [batch, jnp.full((NP - N,), G, jnp.int32)]).reshape(NP // 8, 8).T

    w1k = jnp.kron(jnp.eye(8, dtype=f32), w1p)
    b1r = jnp.tile(b1p, (1, 8))
    w2k = jnp.kron(jnp.eye(8, dtype=f32), w2p)
    b2r = jnp.tile(b2p, (1, 8))

    out16 = pl.pallas_call(
        _post_body,
        grid=(1,),
        in_specs=[pl.BlockSpec((2, NP // 8, 128), lambda i: (0, 0, 0)),
                  pl.BlockSpec((8, NP // 8), lambda i: (0, 0)),
                  pl.BlockSpec((128, 128), lambda i: (0, 0)),
                  pl.BlockSpec((1, 128), lambda i: (0, 0)),
                  pl.BlockSpec((128, 128), lambda i: (0, 0)),
                  pl.BlockSpec((1, 128), lambda i: (0, 0)),
                  pl.BlockSpec((HP, HP), lambda i: (0, 0)),
                  pl.BlockSpec((1, HP), lambda i: (0, 0)),
                  pl.BlockSpec((HP, HP), lambda i: (0, 0)),
                  pl.BlockSpec((1, HP), lambda i: (0, 0))],
        out_specs=pl.BlockSpec((G, HP), lambda i: (0, 0)),
        out_shape=jax.ShapeDtypeStruct((G, HP), f32),
    )(acc2, bat2, w1k, b1r, w2k, b2r, w3p, b3p, w4p, b4p)

    return out16[:, :1]


# speedup vs baseline: 1.3545x; 1.1064x over previous
import functools

import jax
import jax.numpy as jnp
from jax import lax
from jax.experimental import pallas as pl
from jax.experimental.pallas import tpu as pltpu
from jax.experimental.pallas import tpu_sc as plsc

N = 10000
E = 320000
D = 128
DE = 16
H = 10
HP = 16
G = 64

CH = 128
NW = 32
KCW = 80
NCHUNK = KCW * NW
EP = NCHUNK * CH
NP = 10240
RPT = NP // 16

RB = 2048
NB = NP // RB

NBLK = 2000
EBLK = 16000
DR = EBLK // 8
G_ROWS = 400
QPB = DR // 16
NCR = E // CH


def _nproj_body(x_ref, w_ref, o_ref):
    o_ref[...] = jnp.dot(x_ref[...], w_ref[...],
                         preferred_element_type=jnp.float32, precision=jax.lax.Precision.HIGHEST)


def _eproj_body(x_ref, w_ref, b_ref, o_ref):
    for g in range(DR // G_ROWS):
        parts = [x_ref[pl.ds(DR * j + G_ROWS * g, G_ROWS), :]
                 for j in range(8)]
        x128 = jnp.concatenate(parts, axis=1)
        y = jnp.dot(x128, w_ref[...], preferred_element_type=jnp.float32,
                    precision=jax.lax.Precision.HIGHEST) + b_ref[...]
        o_ref[pl.ds(G_ROWS * g, G_ROWS), :] = y


_sc_mesh = plsc.VectorSubcoreMesh(core_axis_name="c", subcore_axis_name="s")


NBUF = 4
TOUT = KCW // NBUF


@functools.partial(
    pl.kernel,
    mesh=_sc_mesh,
    compiler_params=pltpu.CompilerParams(use_tc_tiling_on_sc=False),
    out_type=jax.ShapeDtypeStruct((2, NP, HP), jnp.float32),
    scratch_types=(
        [pltpu.VMEM((KCW, CH), jnp.int32)] * 2 +
        [pltpu.VMEM((CH // 8, 128), jnp.float32)] * NBUF +
        [pltpu.VMEM((CH, HP), jnp.float32)] * (2 * NBUF) +
        [pltpu.VMEM_SHARED((NP, HP), jnp.float32)] +
        [pltpu.SemaphoreType.DMA] * (3 * NBUF)
    ),
)
def _sc_msg(src_hbm, dst_hbm, eproj_hbm, nproj_hbm, out_hbm, *sc):
    src_v, dst_v = sc[0], sc[1]
    ep = sc[2:2 + NBUF]
    nr = sc[2 + NBUF:2 + 2 * NBUF]
    msg = sc[2 + 2 * NBUF:2 + 3 * NBUF]
    acc_sh = sc[2 + 3 * NBUF]
    esem = sc[3 + 3 * NBUF:3 + 4 * NBUF]
    gsem = sc[3 + 4 * NBUF:3 + 5 * NBUF]
    ssem = sc[3 + 5 * NBUF:3 + 6 * NBUF]

    c = lax.axis_index("c")
    s = lax.axis_index("s")
    wid = s * 2 + c

    def zrow(i, c2):
        msg[0][i, :] = jnp.zeros((HP,), jnp.float32)
        return c2

    lax.fori_loop(0, CH, zrow, 0, unroll=8)

    def zcopy(k, c2):
        pltpu.sync_copy(msg[0], acc_sh.at[pl.ds(s * RPT + k * CH, CH)])
        return c2

    lax.fori_loop(0, RPT // CH, zcopy, 0)
    plsc.subcore_barrier()

    base = wid * KCW
    pltpu.sync_copy(src_hbm.at[pl.ds(base, KCW)], src_v)
    pltpu.sync_copy(dst_hbm.at[pl.ds(base, KCW)], dst_v)

    EPR = CH // 8

    def fetch(q, b):
        off = jnp.minimum((base + q) * EPR, (NCR - 1) * EPR)
        pltpu.async_copy(eproj_hbm.at[pl.ds(off, EPR)], ep[b], esem[b])
        pltpu.async_copy(nproj_hbm.at[src_v.at[q]], nr[b], gsem[b])

    def body(t, b, first, last):
        q = t * NBUF + b
        pltpu.make_async_copy(eproj_hbm.at[pl.ds(0, EPR)], ep[b],
                              esem[b]).wait()
        pltpu.make_async_copy(nproj_hbm.at[src_v.at[0]], nr[b],
                              gsem[b]).wait()
        if not first:
            pltpu.make_async_copy(msg[b], acc_sh.at[dst_v.at[0]],
                                  ssem[b]).wait()

        def row(r, c2):
            for u in range(8):
                i = 8 * r + u
                msg[b][i, :] = jnp.maximum(
                    nr[b][i, :] + ep[b][r, pl.ds(16 * u, 16)], 0.0)
            return c2

        lax.fori_loop(0, CH // 8, row, 0, unroll=2)
        pltpu.async_copy(msg[b], acc_sh.at[dst_v.at[q]], ssem[b], add=True)
        if not last:
            fetch(q + NBUF, b)

    for b in range(NBUF):
        fetch(b, b)
    for b in range(NBUF):
        body(0, b, first=True, last=False)

    def steady(t, carry):
        for b in range(NBUF):
            body(t, b, first=False, last=False)
        return carry

    lax.fori_loop(1, TOUT - 1, steady, 0)
    for b in range(NBUF):
        body(TOUT - 1, b, first=False, last=True)
    for b in range(NBUF):
        pltpu.make_async_copy(msg[b], acc_sh.at[dst_v.at[0]], ssem[b]).wait()

    plsc.subcore_barrier()
    pltpu.sync_copy(acc_sh.at[pl.ds(s * RPT, RPT)],
                    out_hbm.at[c].at[pl.ds(s * RPT, RPT)])


def _post_body(acc_ref, bat_ref, w1_ref, b1_ref, w2_ref, b2_ref,
               w3_ref, b3_ref, w4_ref, b4_ref, o_ref):
    hp = jnp.float32
    x = acc_ref[0] + acc_ref[1]
    x = jnp.tanh(jnp.dot(x, w1_ref[...],
                         preferred_element_type=hp,
                         precision=jax.lax.Precision.HIGHEST) + b1_ref[...])
    x = jnp.tanh(jnp.dot(x, w2_ref[...],
                         preferred_element_type=hp,
                         precision=jax.lax.Precision.HIGHEST) + b2_ref[...])
    giota = lax.broadcasted_iota(jnp.int32, (G, NP // 8), 0)
    selr = lax.broadcasted_iota(jnp.int32, (128, HP), 0)
    selh = lax.broadcasted_iota(jnp.int32, (128, HP), 1)
    seg = jnp.zeros((G, HP), hp)
    for j in range(8):
        ohj = (bat_ref[j, :][None, :] == giota).astype(hp)
        mj = jnp.dot(ohj, x, preferred_element_type=hp,
                     precision=jax.lax.Precision.HIGHEST)
        selj = (selr == selh + 16 * j).astype(hp)
        seg = seg + jnp.dot(mj, selj, preferred_element_type=hp,
                            precision=jax.lax.Precision.HIGHEST)
    y = jnp.tanh(jnp.dot(seg, w3_ref[...], preferred_element_type=hp,
                         precision=jax.lax.Precision.HIGHEST) + b3_ref[...])
    o_ref[...] = jnp.dot(y, w4_ref[...], preferred_element_type=hp,
                         precision=jax.lax.Precision.HIGHEST) + b4_ref[...]


def kernel(edge_index, node_attr, edge_attr, batch,
           W_mpl, b_mpl, W1, b1, W2, b2, W3, b3, W4, b4):
    f32 = jnp.float32

    wn = jnp.zeros((D, HP), f32).at[:, :H].set(W_mpl[:D])
    we = jnp.zeros((DE, HP), f32).at[:, :H].set(W_mpl[D:])
    bm = jnp.zeros((1, HP), f32).at[0, :H].set(b_mpl)
    w1p = jnp.zeros((HP, HP), f32).at[:H, :H].set(W1)
    b1p = jnp.zeros((1, HP), f32).at[0, :H].set(b1)
    w2p = jnp.zeros((HP, HP), f32).at[:H, :5].set(W2)
    b2p = jnp.zeros((1, HP), f32).at[0, :5].set(b2)
    w3p = jnp.zeros((HP, HP), f32).at[:5, :5].set(W3)
    b3p = jnp.zeros((1, HP), f32).at[0, :5].set(b3)
    w4p = jnp.zeros((HP, HP), f32).at[:5, :1].set(W4)
    b4p = jnp.zeros((1, HP), f32).at[0, :1].set(b4)

    def _chunked(ids, fill):
        r = (ids.reshape(E // EBLK, 8, QPB, 16)
             .transpose(0, 2, 3, 1).reshape(NCR, CH))
        return jnp.concatenate(
            [r, jnp.full((NCHUNK - NCR, CH), fill, jnp.int32)])

    src = _chunked(edge_index[0], 0)
    dst = _chunked(edge_index[1], N)

    nproj = pl.pallas_call(
        _nproj_body,
        grid=(N // NBLK,),
        in_specs=[pl.BlockSpec((NBLK, D), lambda i: (i, 0)),
                  pl.BlockSpec((D, HP), lambda i: (0, 0))],
        out_specs=pl.BlockSpec((NBLK, HP), lambda i: (i, 0)),
        out_shape=jax.ShapeDtypeStruct((N, HP), f32),
    )(node_attr, wn)

    we8 = jnp.kron(jnp.eye(8, dtype=f32), we)
    bm8 = jnp.tile(bm, (1, 8))
    eproj = pl.pallas_call(
        _eproj_body,
        grid=(E // EBLK,),
        in_specs=[pl.BlockSpec((EBLK, DE), lambda i: (i, 0)),
                  pl.BlockSpec((128, 128), lambda i: (0, 0)),
                  pl.BlockSpec((1, 128), lambda i: (0, 0))],
        out_specs=pl.BlockSpec((DR, 128), lambda i: (i, 0)),
        out_shape=jax.ShapeDtypeStruct((E // 8, 128), f32),
    )(edge_attr, we8, bm8)

    acc = _sc_msg(src, dst, eproj, nproj)
    acc2 = acc.reshape(2, NP // 8, 128)

    bat2 = jnp.concatenate(
        [batch, jnp.full((NP - N,), G, jnp.int32)]).reshape(NP // 8, 8).T

    w1k = jnp.kron(jnp.eye(8, dtype=f32), w1p)
    b1r = jnp.tile(b1p, (1, 8))
    w2k = jnp.kron(jnp.eye(8, dtype=f32), w2p)
    b2r = jnp.tile(b2p, (1, 8))

    out16 = pl.pallas_call(
        _post_body,
        grid=(1,),
        in_specs=[pl.BlockSpec((2, NP // 8, 128), lambda i: (0, 0, 0)),
                  pl.BlockSpec((8, NP // 8), lambda i: (0, 0)),
                  pl.BlockSpec((128, 128), lambda i: (0, 0)),
                  pl.BlockSpec((1, 128), lambda i: (0, 0)),
                  pl.BlockSpec((128, 128), lambda i: (0, 0)),
                  pl.BlockSpec((1, 128), lambda i: (0, 0)),
                  pl.BlockSpec((HP, HP), lambda i: (0, 0)),
                  pl.BlockSpec((1, HP), lambda i: (0, 0)),
                  pl.BlockSpec((HP, HP), lambda i: (0, 0)),
                  pl.BlockSpec((1, HP), lambda i: (0, 0))],
        out_specs=pl.BlockSpec((G, HP), lambda i: (0, 0)),
        out_shape=jax.ShapeDtypeStruct((G, HP), f32),
    )(acc2, bat2, w1k, b1r, w2k, b2r, w3p, b3p, w4p, b4p)

    return out16[:, :1]
